# baseline (speedup 1.0000x reference)
import functools

import jax
import jax.numpy as jnp
from jax import lax
from jax.experimental import pallas as pl
from jax.experimental.pallas import tpu as pltpu
from jax.experimental.pallas import tpu_sc as plsc

B = 2048
IN = 256
OUT = 128
LANE = 128
NCHUNK = B // LANE
OGRP = 8

SC_NC = 2
SC_NS = 16
SCL = 16
NW = SC_NC * SC_NS
CPT = OUT // NW
NV = IN // SCL


def _xor_perm1(a, j):
    iota = lax.broadcasted_iota(jnp.int32, a.shape, 1)
    bit = (iota & j) != 0
    up = jnp.roll(a, j, axis=1)
    dn = jnp.roll(a, -j, axis=1)
    return jnp.where(bit, up, dn)


def _score_body(xT_ref, tT_ref, wT_ref, score_ref, sw_ref):
    f32 = jnp.float32
    ones_col = jnp.ones((LANE, 1), f32)

    dacc = xT_ref[:, 0:LANE]
    for c in range(1, NCHUNK):
        dacc = dacc + xT_ref[:, c * LANE:(c + 1) * LANE]
    denomT = lax.dot_general(ones_col, dacc, (((0,), (1,)), ((), ())),
                             preferred_element_type=f32)

    def gbody(g, carry):
        o0 = g * OGRP
        for ih in range(2):
            rs = slice(ih * (IN // 2), (ih + 1) * (IN // 2))
            accs = [None] * OGRP
            for c in range(NCHUNK):
                cs = slice(c * LANE, (c + 1) * LANE)
                xc = xT_ref[rs, cs]
                t8 = tT_ref[pl.ds(o0, OGRP), cs]
                for r in range(OGRP):
                    trow = lax.slice(t8, (r, 0), (r + 1, LANE))
                    m = jnp.minimum(xc, trow)
                    accs[r] = m if c == 0 else accs[r] + m
            srows = [lax.dot_general(ones_col, accs[r], (((0,), (1,)), ((), ())),
                                     preferred_element_type=f32)
                     for r in range(OGRP)]
            sblkT = jnp.concatenate(srows, axis=0)
            score_ref[pl.ds(o0, OGRP), rs] = sblkT
        return carry

    lax.fori_loop(0, OUT // OGRP, gbody, 0)

    sT = score_ref[...]
    score_ref[...] = jnp.where(denomT == 0.0, jnp.float32(1.0), sT / denomT)

    iota1 = lax.broadcasted_iota(jnp.int32, (OUT, IN), 1)
    sw = wT_ref[...]
    for k in [2, 4, 8, 16, 32, 64, 128, 256]:
        j = k // 2
        while j >= 1:
            swp = _xor_perm1(sw, j)
            is_lower = (iota1 & j) == 0
            d = (iota1 & k) == 0
            hi = jnp.maximum(sw, swp)
            lo = jnp.minimum(sw, swp)
            sw = jnp.where(is_lower == d, hi, lo)
            j //= 2
    sw_ref[...] = sw


def _sc_clean_kv(ks, vs):
    r = len(ks)
    if r == 1:
        k, v = plsc.sort_key_val(ks[0], vs[0], descending=True)
        return [k], [v]
    h = r // 2
    lo_k, lo_v, hi_k, hi_v = [], [], [], []
    for j in range(h):
        sel = ks[j] >= ks[j + h]
        lo_k.append(jnp.where(sel, ks[j], ks[j + h]))
        lo_v.append(jnp.where(sel, vs[j], vs[j + h]))
        hi_k.append(jnp.where(sel, ks[j + h], ks[j]))
        hi_v.append(jnp.where(sel, vs[j + h], vs[j]))
    ak, av = _sc_clean_kv(lo_k, lo_v)
    bk, bv = _sc_clean_kv(hi_k, hi_v)
    return ak + bk, av + bv


def _sc_merge_kv(ak, av, bk, bv):
    r = len(ak)
    bk = [lax.rev(x, (0,)) for x in reversed(bk)]
    bv = [lax.rev(x, (0,)) for x in reversed(bv)]
    lo_k, lo_v, hi_k, hi_v = [], [], [], []
    for j in range(r):
        sel = ak[j] >= bk[j]
        lo_k.append(jnp.where(sel, ak[j], bk[j]))
        lo_v.append(jnp.where(sel, av[j], bv[j]))
        hi_k.append(jnp.where(sel, bk[j], ak[j]))
        hi_v.append(jnp.where(sel, bv[j], av[j]))
    ck, cv = _sc_clean_kv(lo_k, lo_v)
    dk, dv = _sc_clean_kv(hi_k, hi_v)
    return ck + dk, cv + dv


def _sc_sort_col_kv(ks, vs):
    runs = [tuple([x] for x in plsc.sort_key_val(k, v, descending=True))
            for k, v in zip(ks, vs)]
    while len(runs) > 1:
        runs = [_sc_merge_kv(*runs[i], *runs[i + 1])
                for i in range(0, len(runs), 2)]
    return runs[0]


def _sc_sort_body(scoreT_hbm, wT_hbm, swT_hbm, out_hbm, sv, wv, swv, pv):
    cid = lax.axis_index("c")
    sid = lax.axis_index("s")
    wid = sid * SC_NC + cid
    row0 = wid * CPT
    pltpu.sync_copy(scoreT_hbm.at[pl.ds(row0, CPT)], sv)
    pltpu.sync_copy(wT_hbm.at[pl.ds(row0, CPT)], wv)
    pltpu.sync_copy(swT_hbm.at[pl.ds(row0, CPT)], swv)

    acc = jnp.zeros((SCL,), jnp.float32)
    for c in range(CPT):
        ks = [sv[c, pl.ds(v * SCL, SCL)] for v in range(NV)]
        vs = [wv[c, pl.ds(v * SCL, SCL)] for v in range(NV)]
        _, tw = _sc_sort_col_kv(ks, vs)
        for v in range(NV):
            d = swv[c, pl.ds(v * SCL, SCL)] - tw[v]
            acc = acc + d * d
    pv[...] = acc
    pltpu.sync_copy(pv, out_hbm.at[wid])


_sc_sort = functools.partial(
    pl.kernel,
    out_type=jax.ShapeDtypeStruct((NW, SCL), jnp.float32),
    mesh=plsc.VectorSubcoreMesh(core_axis_name="c", subcore_axis_name="s"),
    compiler_params=pltpu.CompilerParams(needs_layout_passes=False),
    scratch_types=[
        pltpu.VMEM((CPT, IN), jnp.float32),
        pltpu.VMEM((CPT, IN), jnp.float32),
        pltpu.VMEM((CPT, IN), jnp.float32),
        pltpu.VMEM((SCL,), jnp.float32),
    ],
)(_sc_sort_body)


@jax.jit
def _run(x, t, w):
    xT = x.T
    tT = t.T
    wT = w.T
    scoreT, swT = pl.pallas_call(
        _score_body,
        out_shape=[
            jax.ShapeDtypeStruct((OUT, IN), jnp.float32),
            jax.ShapeDtypeStruct((OUT, IN), jnp.float32),
        ],
    )(xT, tT, wT)
    partials = _sc_sort(scoreT, wT, swT)
    return jnp.sum(partials) / jnp.float32(IN * OUT)


def kernel(x, y, t, w):
    del y
    return _run(x, t, w)


# speedup vs baseline: 1.5060x; 1.5060x over previous
import functools

import jax
import jax.numpy as jnp
from jax import lax
from jax.experimental import pallas as pl
from jax.experimental.pallas import tpu as pltpu

B = 2048
IN = 256
OUT = 128
LANE = 128
NCHUNK = B // LANE
OGRP = 8


def _xor_perm1(a, j):
    iota = lax.broadcasted_iota(jnp.int32, a.shape, 1)
    bit = (iota & j) != 0
    up = jnp.roll(a, j, axis=1)
    dn = jnp.roll(a, -j, axis=1)
    return jnp.where(bit, up, dn)


def _loss_body(xT_ref, tT_ref, wT_ref, out_ref, sT_ref):
    f32 = jnp.float32
    ones_col = jnp.ones((LANE, 1), f32)

    dacc = xT_ref[:, 0:LANE]
    for c in range(1, NCHUNK):
        dacc = dacc + xT_ref[:, c * LANE:(c + 1) * LANE]
    denomT = lax.dot_general(ones_col, dacc, (((0,), (1,)), ((), ())),
                             preferred_element_type=f32)

    def gbody(g, carry):
        o0 = g * OGRP
        for ih in range(2):
            rs = slice(ih * (IN // 2), (ih + 1) * (IN // 2))
            accs = [None] * OGRP
            for c in range(NCHUNK):
                cs = slice(c * LANE, (c + 1) * LANE)
                xc = xT_ref[rs, cs]
                t8 = tT_ref[pl.ds(o0, OGRP), cs]
                for r in range(OGRP):
                    trow = lax.slice(t8, (r, 0), (r + 1, LANE))
                    m = jnp.minimum(xc, trow)
                    accs[r] = m if c == 0 else accs[r] + m
            srows = [lax.dot_general(ones_col, accs[r], (((0,), (1,)), ((), ())),
                                     preferred_element_type=f32)
                     for r in range(OGRP)]
            sblkT = jnp.concatenate(srows, axis=0)
            sT_ref[pl.ds(o0, OGRP), rs] = sblkT
        return carry

    lax.fori_loop(0, OUT // OGRP, gbody, 0)

    sT = sT_ref[...]
    scoreT = jnp.where(denomT == 0.0, jnp.float32(1.0), sT / denomT)

    iota1 = lax.broadcasted_iota(jnp.int32, (OUT, IN), 1)
    key = scoreT
    pay = wT_ref[...]
    sw = wT_ref[...]
    for k in [2, 4, 8, 16, 32, 64, 128, 256]:
        j = k // 2
        while j >= 1:
            kp = _xor_perm1(key, j)
            pp = _xor_perm1(pay, j)
            swp = _xor_perm1(sw, j)
            is_lower = (iota1 & j) == 0
            before = (key > kp) | (is_lower & (key == kp))
            pbits = iota1 & (k + j)
            flip = (pbits == k) | (pbits == j)
            keep = before != flip
            key = jnp.where(keep, key, kp)
            pay = jnp.where(keep, pay, pp)
            d = (iota1 & k) == 0
            hi = jnp.maximum(sw, swp)
            lo = jnp.minimum(sw, swp)
            sw = jnp.where(is_lower == d, hi, lo)
            j //= 2
    target_w = pay
    sorted_w = sw

    diff = sorted_w - target_w
    sq = diff * diff
    total = jnp.sum(jnp.sum(sq, axis=0, keepdims=True), axis=1, keepdims=True)
    out_ref[...] = total / jnp.float32(IN * OUT)


@functools.partial(jax.jit, static_argnames=("interpret",))
def _run(x, t, w, interpret=False):
    xT = x.T
    tT = t.T
    wT = w.T
    out = pl.pallas_call(
        _loss_body,
        out_shape=jax.ShapeDtypeStruct((1, 1), jnp.float32),
        scratch_shapes=[pltpu.VMEM((OUT, IN), jnp.float32)],
        interpret=interpret,
    )(xT, tT, wT)
    return out[0, 0]


def kernel(x, y, t, w):
    del y
    return _run(x, t, w)
